# Initial kernel scaffold; baseline (speedup 1.0000x reference)
import jax
import jax.numpy as jnp
from jax.experimental import pallas as pl


def kernel(x, a, h_state, kernel, attn_self, attn_neighs, gat_bias, b_u, b_r, b_c, W_u, W_r, W_c, R_p):
    raise NotImplementedError("write your pallas kernel here")


# speedup vs baseline: 1.2667x; 1.2667x over previous
import functools

import jax
import jax.numpy as jnp
from jax.experimental import pallas as pl

N = 4096
F = 128
H = 2
C = 64
D = 64
HC = H * C
R = 256
NEG = -1e10


def _feat_body(x_ref, wk_ref, ss_ref, sn_ref, xk_ref, af_ref):
    xk = jnp.dot(x_ref[...], wk_ref[...], preferred_element_type=jnp.float32)
    xk_ref[...] = xk
    afs = jnp.dot(xk, ss_ref[...], preferred_element_type=jnp.float32)
    afn = jnp.dot(xk, sn_ref[...], preferred_element_type=jnp.float32)
    af_ref[...] = jnp.concatenate([afs, afn], axis=1)


def _gat_gru_body(a_ref, xkf_ref, af_full_ref, af_blk_ref, h_ref,
                  bu_ref, br_ref, bc_ref, gb_ref, wu_ref, wr_ref, wc_ref,
                  h1_ref):
    i = pl.program_id(0)
    a_blk = a_ref[...]
    xk_f = xkf_ref[...]
    h_b = h_ref[...]

    col = jax.lax.broadcasted_iota(jnp.int32, (R, N), 1)
    row_g = i * R + jax.lax.broadcasted_iota(jnp.int32, (R, N), 0)
    keep = (a_blk != 0.0) | (col == row_g)

    outs = []
    for h in range(H):
        afs_h = af_blk_ref[:, h:h + 1]
        afn_h = af_full_ref[:, H + h:H + h + 1].reshape(1, N)
        lg = afs_h + afn_h
        lg = jnp.where(lg >= 0.0, lg, 0.2 * lg)
        lg = jnp.where(keep, lg, NEG)
        m = jnp.max(lg, axis=1, keepdims=True)
        e = jnp.exp(lg - m)
        s = jnp.sum(e, axis=1, keepdims=True)
        attn = e / s
        outs.append(jnp.dot(attn, xk_f[:, h * C:(h + 1) * C],
                            preferred_element_type=jnp.float32))
    conv = jnp.concatenate(outs, axis=1) + gb_ref[...]

    wu = wu_ref[...]
    wr = wr_ref[...]
    wc = wc_ref[...]
    u = jax.nn.sigmoid(bu_ref[...]
                       + jnp.dot(conv, wu[:HC, :], preferred_element_type=jnp.float32)
                       + jnp.dot(h_b, wu[HC:, :], preferred_element_type=jnp.float32))
    r = jax.nn.sigmoid(br_ref[...]
                       + jnp.dot(conv, wr[:HC, :], preferred_element_type=jnp.float32)
                       + jnp.dot(h_b, wr[HC:, :], preferred_element_type=jnp.float32))
    c = jnp.tanh(bc_ref[...]
                 + jnp.dot(conv, wc[:HC, :], preferred_element_type=jnp.float32)
                 + jnp.dot(r * h_b, wc[HC:, :], preferred_element_type=jnp.float32))
    h1_ref[...] = u * h_b + (1.0 - u) * c


def _decode_body(hb_ref, hf_ref, rp_ref, a_ref):
    hr = jnp.dot(hb_ref[...], rp_ref[...], preferred_element_type=jnp.float32)
    a_ref[...] = jax.lax.dot_general(
        hr, hf_ref[...], (((1,), (1,)), ((), ())),
        preferred_element_type=jnp.float32)


@jax.jit
def kernel(x, a, h_state, kernel, attn_self, attn_neighs, gat_bias,
           b_u, b_r, b_c, W_u, W_r, W_c, R_p):
    x2 = x.reshape(N, F)
    a2 = a.reshape(N, N)
    h2 = h_state.reshape(N, D)
    wk = kernel.reshape(F, HC)
    hsel = (jnp.arange(HC, dtype=jnp.int32) // C)[:, None] \
        == jnp.arange(H, dtype=jnp.int32)[None, :]
    ss = jnp.where(hsel, jnp.tile(attn_self[:, :, 0], (H, 1)), 0.0)
    sn = jnp.where(hsel, jnp.tile(attn_neighs[:, :, 0], (H, 1)), 0.0)
    gb = gat_bias.reshape(1, HC)

    xk, af = pl.pallas_call(
        _feat_body,
        out_shape=(jax.ShapeDtypeStruct((N, HC), jnp.float32),
                   jax.ShapeDtypeStruct((N, 2 * H), jnp.float32)),
    )(x2, wk, ss, sn)

    nblk = N // R
    full = lambda i: (0, 0)
    blk = lambda i: (i, 0)
    h1 = pl.pallas_call(
        _gat_gru_body,
        grid=(nblk,),
        in_specs=[
            pl.BlockSpec((R, N), blk),
            pl.BlockSpec((N, HC), full),
            pl.BlockSpec((N, 2 * H), full),
            pl.BlockSpec((R, 2 * H), blk),
            pl.BlockSpec((R, D), blk),
            pl.BlockSpec((R, 1), blk),
            pl.BlockSpec((R, 1), blk),
            pl.BlockSpec((R, 1), blk),
            pl.BlockSpec((1, HC), full),
            pl.BlockSpec((HC + D, D), full),
            pl.BlockSpec((HC + D, D), full),
            pl.BlockSpec((HC + D, D), full),
        ],
        out_specs=pl.BlockSpec((R, D), blk),
        out_shape=jax.ShapeDtypeStruct((N, D), jnp.float32),
    )(a2, xk, af, af, h2, b_u, b_r, b_c, gb, W_u, W_r, W_c)

    A = pl.pallas_call(
        _decode_body,
        grid=(nblk,),
        in_specs=[
            pl.BlockSpec((R, D), blk),
            pl.BlockSpec((N, D), full),
            pl.BlockSpec((D, D), full),
        ],
        out_specs=pl.BlockSpec((R, N), blk),
        out_shape=jax.ShapeDtypeStruct((N, N), jnp.float32),
    )(h1, h1, R_p)

    return (A.reshape(1, N, N), h1.reshape(1, N, D))


# speedup vs baseline: 1.9858x; 1.5677x over previous
import jax
import jax.numpy as jnp
from jax.experimental import pallas as pl

N = 4096
F = 128
H = 2
C = 64
D = 64
HC = H * C
R = 256


def _feat_body(x_ref, wk_ref, ss4_ref, sn4t_ref,
               aug0_ref, aug1_ref, afse_ref, afne_ref):
    xk = jnp.dot(x_ref[...], wk_ref[...], preferred_element_type=jnp.float32)
    af4 = jnp.dot(xk, ss4_ref[...], preferred_element_type=jnp.float32)
    afse_ref[...] = jnp.exp(af4).astype(jnp.bfloat16)
    afn4 = jax.lax.dot_general(sn4t_ref[...], xk, (((1,), (1,)), ((), ())),
                               preferred_element_type=jnp.float32)
    q = jnp.exp(afn4)
    afne_ref[...] = jnp.concatenate(
        [q, jnp.zeros((12, N), jnp.float32)], axis=0).astype(jnp.bfloat16)
    ones = jnp.ones((N, 1), jnp.float32)
    aug0_ref[...] = jnp.concatenate([xk[:, :C], ones], axis=1).astype(jnp.bfloat16)
    aug1_ref[...] = jnp.concatenate([xk[:, C:], ones], axis=1).astype(jnp.bfloat16)


def _gat_gru_body(a_ref, aug0_ref, aug1_ref, afse_ref, afne_ref, h_ref,
                  bu_ref, br_ref, bc_ref, gb0_ref, gb1_ref,
                  wu_ref, wr_ref, wc_ref, h1_ref):
    i = pl.program_id(0)
    a_blk = a_ref[...]
    col = jax.lax.broadcasted_iota(jnp.int32, (R, N), 1)
    row_g = i * R + jax.lax.broadcasted_iota(jnp.int32, (R, N), 0)
    keep = (a_blk != 0.0) | (col == row_g)

    convs = []
    for h, aug_ref, gb_ref in ((0, aug0_ref, gb0_ref), (1, aug1_ref, gb1_ref)):
        p1 = afse_ref[:, h:h + 1]
        p2 = afse_ref[:, 2 + h:3 + h]
        q1 = afne_ref[h:h + 1, :]
        q2 = afne_ref[2 + h:3 + h, :]
        s = p1 * q1
        t = p2 * q2
        e = jnp.where(s >= 1.0, s, t)
        w = jnp.where(keep, e, jnp.bfloat16(0.0))
        agg = jnp.dot(w, aug_ref[...], preferred_element_type=jnp.float32)
        convs.append(agg[:, :C] / agg[:, C:C + 1] + gb_ref[...])
    c0, c1 = convs

    h_b = h_ref[...]
    wu = wu_ref[...]
    wr = wr_ref[...]
    wc = wc_ref[...]

    def mm3(w, a0, a1, a2):
        return (jnp.dot(a0, w[:C, :], preferred_element_type=jnp.float32)
                + jnp.dot(a1, w[C:HC, :], preferred_element_type=jnp.float32)
                + jnp.dot(a2, w[HC:, :], preferred_element_type=jnp.float32))

    u = jax.nn.sigmoid(bu_ref[...] + mm3(wu, c0, c1, h_b))
    r = jax.nn.sigmoid(br_ref[...] + mm3(wr, c0, c1, h_b))
    c = jnp.tanh(bc_ref[...] + mm3(wc, c0, c1, r * h_b))
    h1_ref[...] = u * h_b + (1.0 - u) * c


def _decode_body(hb_ref, hf_ref, rp_ref, a_ref):
    hr = jnp.dot(hb_ref[...], rp_ref[...], preferred_element_type=jnp.float32)
    a_ref[...] = jax.lax.dot_general(
        hr, hf_ref[...], (((1,), (1,)), ((), ())),
        preferred_element_type=jnp.float32)


@jax.jit
def kernel(x, a, h_state, kernel, attn_self, attn_neighs, gat_bias,
           b_u, b_r, b_c, W_u, W_r, W_c, R_p):
    x2 = x.reshape(N, F)
    a2 = a.reshape(N, N)
    h2 = h_state.reshape(N, D)
    wk = kernel.reshape(F, HC)
    hsel = (jnp.arange(HC, dtype=jnp.int32) // C)[:, None] \
        == jnp.arange(H, dtype=jnp.int32)[None, :]
    ss = jnp.where(hsel, jnp.tile(attn_self[:, :, 0], (H, 1)), 0.0)
    sn = jnp.where(hsel, jnp.tile(attn_neighs[:, :, 0], (H, 1)), 0.0)
    ss4 = jnp.concatenate([ss, 0.2 * ss], axis=1)
    sn4t = jnp.concatenate([sn.T, 0.2 * sn.T], axis=0)
    gb0 = gat_bias[:C].reshape(1, C)
    gb1 = gat_bias[C:].reshape(1, C)

    aug0, aug1, afse, afne = pl.pallas_call(
        _feat_body,
        out_shape=(jax.ShapeDtypeStruct((N, C + 1), jnp.bfloat16),
                   jax.ShapeDtypeStruct((N, C + 1), jnp.bfloat16),
                   jax.ShapeDtypeStruct((N, 4), jnp.bfloat16),
                   jax.ShapeDtypeStruct((16, N), jnp.bfloat16)),
    )(x2, wk, ss4, sn4t)

    nblk = N // R
    full = lambda i: (0, 0)
    blk = lambda i: (i, 0)
    h1 = pl.pallas_call(
        _gat_gru_body,
        grid=(nblk,),
        in_specs=[
            pl.BlockSpec((R, N), blk),
            pl.BlockSpec((N, C + 1), full),
            pl.BlockSpec((N, C + 1), full),
            pl.BlockSpec((R, 4), blk),
            pl.BlockSpec((16, N), full),
            pl.BlockSpec((R, D), blk),
            pl.BlockSpec((R, 1), blk),
            pl.BlockSpec((R, 1), blk),
            pl.BlockSpec((R, 1), blk),
            pl.BlockSpec((1, C), full),
            pl.BlockSpec((1, C), full),
            pl.BlockSpec((HC + D, D), full),
            pl.BlockSpec((HC + D, D), full),
            pl.BlockSpec((HC + D, D), full),
        ],
        out_specs=pl.BlockSpec((R, D), blk),
        out_shape=jax.ShapeDtypeStruct((N, D), jnp.float32),
    )(a2, aug0, aug1, afse, afne, h2, b_u, b_r, b_c, gb0, gb1, W_u, W_r, W_c)

    A = pl.pallas_call(
        _decode_body,
        grid=(nblk,),
        in_specs=[
            pl.BlockSpec((R, D), blk),
            pl.BlockSpec((N, D), full),
            pl.BlockSpec((D, D), full),
        ],
        out_specs=pl.BlockSpec((R, N), blk),
        out_shape=jax.ShapeDtypeStruct((N, N), jnp.float32),
    )(h1, h1, R_p)

    return (A.reshape(1, N, N), h1.reshape(1, N, D))
